# baseline (speedup 1.0000x reference)
import functools

import jax
import jax.numpy as jnp
from jax import lax
from jax.experimental import pallas as pl
from jax.experimental.pallas import tpu as pltpu
from jax.experimental.pallas import tpu_sc as plsc

NK = 10000
NP = 10240
E = 160000
EPS = 10368
EP = 16 * EPS
ED = 163840
NCH = EPS // 128
D = 256
H = 128
NG = 64
R = 2048

_mesh = plsc.VectorSubcoreMesh(core_axis_name="c", subcore_axis_name="s")


@functools.partial(
    pl.kernel,
    out_type=jax.ShapeDtypeStruct((2 * NP, 16), jnp.float32),
    mesh=_mesh,
    scratch_types=[
        pltpu.VMEM_SHARED((NP, 16), jnp.float32),
        pltpu.VMEM((640, 16), jnp.float32),
        pltpu.VMEM((128, 16), jnp.float32),
        pltpu.VMEM((40, 128), jnp.int32),
        pltpu.SemaphoreType.DMA,
    ],
)
def _sc_deg(dstd_hbm, out_hbm, acc, zbuf, vals, didx, sem):
    c = lax.axis_index("c")
    s = lax.axis_index("s")
    zero16 = jnp.zeros((16,), jnp.float32)
    e0 = jnp.where(lax.iota(jnp.int32, 16) == 0, 1.0, 0.0).astype(jnp.float32)

    @pl.loop(0, 640)
    def _(j):
        zbuf[j, :] = zero16

    @pl.loop(0, 128)
    def _(j):
        vals[j, :] = e0

    w = c * 16 + s
    pltpu.sync_copy(dstd_hbm.at[pl.ds(w * 40, 40)], didx)
    pltpu.sync_copy(zbuf, acc.at[pl.ds(s * 640, 640)])
    plsc.subcore_barrier()

    @pl.loop(0, 40)
    def _(i):
        pltpu.async_copy(vals, acc.at[didx.at[i]], sem, add=True)

    @pl.loop(0, 40)
    def _(i):
        pltpu.make_async_copy(vals, acc.at[didx.at[i]], sem).wait()

    plsc.subcore_barrier()
    pltpu.sync_copy(acc.at[pl.ds(s * 640, 640)],
                    out_hbm.at[pl.ds(c * NP + s * 640, 640)])


@functools.partial(
    pl.kernel,
    out_type=jax.ShapeDtypeStruct((2 * NP, H), jnp.float32),
    mesh=_mesh,
    scratch_types=[
        pltpu.VMEM_SHARED((NK, H), jnp.float32),
        pltpu.VMEM((3, 2, 128), jnp.int32),
        pltpu.VMEM((3, 128, H), jnp.float32),
        pltpu.SemaphoreType.DMA((3,)),
        pltpu.SemaphoreType.DMA((3,)),
        pltpu.SemaphoreType.DMA((3,)),
    ],
)
def _sc_prop(g_hbm, pidx_hbm, zpad_hbm, out_hbm, acc, pidx, rows, isem, gsem,
             ssem):
    c = lax.axis_index("c")
    s = lax.axis_index("s")
    pltpu.sync_copy(g_hbm.at[pl.ds(c * NP + s * 624, 624)],
                    acc.at[pl.ds(s * 624, 624)])

    @pl.when(s == 0)
    def _():
        pltpu.sync_copy(g_hbm.at[pl.ds(c * NP + 9984, 16)],
                        acc.at[pl.ds(9984, 16)])

    w = c * 16 + s

    def fetch(g2, b):
        pltpu.async_copy(pidx_hbm.at[pl.ds((w * NCH + g2) * 2, 2)],
                         pidx.at[b], isem.at[b])

    def wait_idx(b):
        pltpu.make_async_copy(pidx_hbm.at[pl.ds(0, 2)], pidx.at[b],
                              isem.at[b]).wait()

    def gather(b):
        pltpu.async_copy(g_hbm.at[pidx.at[b, 0]], rows.at[b], gsem.at[b])

    def wait_gather(b):
        pltpu.make_async_copy(g_hbm.at[pidx.at[b, 0]], rows.at[b],
                              gsem.at[b]).wait()

    def scatter(b):
        pltpu.async_copy(rows.at[b], acc.at[pidx.at[b, 1]], ssem.at[b],
                         add=True)

    def wait_scatter(b):
        pltpu.make_async_copy(rows.at[b], acc.at[pidx.at[b, 1]],
                              ssem.at[b]).wait()

    def lap(g, b, first=False, do_f=True, do_g=True):
        b1, b2 = (b + 1) % 3, (b + 2) % 3
        wait_gather(b)
        scatter(b)
        if not first:
            wait_scatter(b2)
        if do_f:
            fetch(g + 2, b2)
        if do_g:
            wait_idx(b1)
            gather(b1)

    fetch(0, 0)
    fetch(1, 1)
    plsc.subcore_barrier()
    wait_idx(0)
    gather(0)
    lap(0, 0, first=True)
    lap(1, 1)
    lap(2, 2)
    lap(3, 0)

    @pl.loop(0, (NCH - 6) // 3)
    def _(t):
        g = 4 + t * 3
        lap(g, 1)
        lap(g + 1, 2)
        lap(g + 2, 0)

    lap(NCH - 2, 1, do_f=False)
    lap(NCH - 1, 2, do_f=False, do_g=False)
    wait_scatter(2)

    plsc.subcore_barrier()
    pltpu.sync_copy(acc.at[pl.ds(s * 624, 624)],
                    out_hbm.at[pl.ds(c * NP + s * 624, 624)])

    @pl.when(s == 0)
    def _():
        pltpu.sync_copy(acc.at[pl.ds(9984, 16)],
                        out_hbm.at[pl.ds(c * NP + 9984, 16)])

    @pl.when(s == 15)
    def _():
        pltpu.sync_copy(zpad_hbm, out_hbm.at[pl.ds(c * NP + NK, NP - NK)])


def _dis_of(d_blk):
    deg = d_blk[0, :, 0:1] + d_blk[1, :, 0:1] + 1.0
    return lax.rsqrt(deg)


def _mm_first_body(x_ref, w_ref, d_ref, o_ref):
    dis = _dis_of(d_ref[...])
    t = jnp.dot(x_ref[...], w_ref[...], preferred_element_type=jnp.float32)
    g = t * dis
    o_ref[0] = g[:, :H]
    o_ref[1] = g[:, H:]


def _mm_mid_body(p_ref, w_ref, d_ref, o_ref):
    pb = p_ref[...]
    dis = _dis_of(d_ref[...])
    h = jnp.concatenate([pb[0], pb[1]], axis=1)
    h = jnp.maximum(h * dis, 0.0)
    t = jnp.dot(h, w_ref[...], preferred_element_type=jnp.float32)
    g = t * dis
    o_ref[0] = g[:, :H]
    o_ref[1] = g[:, H:]


def _mm_first(x_pad, W, degp):
    return pl.pallas_call(
        _mm_first_body,
        grid=(NP // R,),
        in_specs=[
            pl.BlockSpec((R, D), lambda i: (i, 0)),
            pl.BlockSpec((D, D), lambda i: (0, 0)),
            pl.BlockSpec((2, R, 16), lambda i: (0, i, 0)),
        ],
        out_specs=pl.BlockSpec((2, R, H), lambda i: (0, i, 0)),
        out_shape=jax.ShapeDtypeStruct((2, NP, H), jnp.float32),
    )(x_pad, W, degp)


def _mm_mid(P, W, degp):
    return pl.pallas_call(
        _mm_mid_body,
        grid=(NP // R,),
        in_specs=[
            pl.BlockSpec((2, R, H), lambda i: (0, i, 0)),
            pl.BlockSpec((D, D), lambda i: (0, 0)),
            pl.BlockSpec((2, R, 16), lambda i: (0, i, 0)),
        ],
        out_specs=pl.BlockSpec((2, R, H), lambda i: (0, i, 0)),
        out_shape=jax.ShapeDtypeStruct((2, NP, H), jnp.float32),
    )(P, W, degp)


RP = 2000


def _pool_body(p_ref, d_ref, b_ref, hn_ref, hg_ref, sums, counts):
    i = pl.program_id(0)
    pb = p_ref[...]
    dis = _dis_of(d_ref[...])
    h = jnp.concatenate([pb[0], pb[1]], axis=1)
    h = jnp.maximum(h * dis, 0.0)
    hn_ref[...] = h
    bb = b_ref[...]
    gid = lax.broadcasted_iota(jnp.int32, (RP, NG), 1)
    mask = (bb == gid).astype(jnp.float32)
    dn = (((0,), (0,)), ((), ()))
    s_blk = lax.dot_general(mask, h, dn, preferred_element_type=jnp.float32)
    ones = jnp.ones((RP, H), jnp.float32)
    c_blk = lax.dot_general(mask, ones, dn, preferred_element_type=jnp.float32)

    @pl.when(i == 0)
    def _():
        sums[...] = s_blk
        counts[...] = c_blk

    @pl.when(i > 0)
    def _():
        sums[...] += s_blk
        counts[...] += c_blk

    @pl.when(i == NK // RP - 1)
    def _():
        cnt = jnp.maximum(counts[...][:, 0:1], 1.0)
        hg_ref[...] = sums[...] / cnt


def _pool(P, degp, batch_col):
    return pl.pallas_call(
        _pool_body,
        grid=(NK // RP,),
        in_specs=[
            pl.BlockSpec((2, RP, H), lambda i: (0, i, 0)),
            pl.BlockSpec((2, RP, 16), lambda i: (0, i, 0)),
            pl.BlockSpec((RP, 1), lambda i: (i, 0)),
        ],
        out_specs=[
            pl.BlockSpec((RP, D), lambda i: (i, 0)),
            pl.BlockSpec((NG, D), lambda i: (0, 0)),
        ],
        out_shape=[
            jax.ShapeDtypeStruct((NK, D), jnp.float32),
            jax.ShapeDtypeStruct((NG, D), jnp.float32),
        ],
        scratch_shapes=[
            pltpu.VMEM((NG, D), jnp.float32),
            pltpu.VMEM((NG, H), jnp.float32),
        ],
    )(P, degp, batch_col)


def kernel(x, edge_index, edge_weight, batch, W0, W1, W2):
    src = edge_index[0]
    dst = edge_index[1]

    e = jnp.arange(E, EP, dtype=jnp.int32)
    src_p = jnp.concatenate([src, NK + e % (NP - NK)])
    dst_p = jnp.concatenate([dst, e % NK])
    srcm = jnp.stack([src_p, src_p + NP])
    dstm = jnp.broadcast_to(dst_p[None], (2, EP))
    pidx = jnp.stack([srcm.reshape(2, 16, NCH, 128),
                      dstm.reshape(2, 16, NCH, 128)], axis=3)
    pidx = pidx.reshape(2 * 16 * NCH * 2, 128)

    e2 = jnp.arange(E, ED, dtype=jnp.int32)
    dstd = jnp.concatenate([dst, NK + e2 % (NP - NK)])

    x_pad = jnp.pad(x, ((0, NP - NK), (0, 0)))
    batch_col = batch.reshape(NK, 1)
    zpad = jnp.zeros((NP - NK, H), jnp.float32)

    degp = _sc_deg(dstd.reshape(32 * 40, 128)).reshape(2, NP, 16)
    g = _mm_first(x_pad, W0, degp)
    P = _sc_prop(g.reshape(2 * NP, H), pidx, zpad).reshape(2, NP, H)
    g = _mm_mid(P, W1, degp)
    P = _sc_prop(g.reshape(2 * NP, H), pidx, zpad).reshape(2, NP, H)
    g = _mm_mid(P, W2, degp)
    P = _sc_prop(g.reshape(2 * NP, H), pidx, zpad).reshape(2, NP, H)
    h_node, h_graph = _pool(P, degp, batch_col)
    return h_node, h_graph


# speedup vs baseline: 1.2148x; 1.2148x over previous
import functools

import jax
import jax.numpy as jnp
from jax import lax
from jax.experimental import pallas as pl
from jax.experimental.pallas import tpu as pltpu
from jax.experimental.pallas import tpu_sc as plsc

NK = 10000
NP = 10240
E = 160000
EPS = 10240
EP = 16 * EPS
NCH = EPS // 128
NB = 2
D = 256
H = 128
NG = 64
R = 2048

_mesh = plsc.VectorSubcoreMesh(core_axis_name="c", subcore_axis_name="s")


@functools.partial(
    pl.kernel,
    out_type=jax.ShapeDtypeStruct((2 * NP, 16), jnp.float32),
    mesh=_mesh,
    scratch_types=[
        pltpu.VMEM_SHARED((NP, 16), jnp.float32),
        pltpu.VMEM((640, 16), jnp.float32),
        pltpu.VMEM((128, 16), jnp.float32),
        pltpu.VMEM((40, 128), jnp.int32),
        pltpu.SemaphoreType.DMA,
    ],
)
def _sc_deg(dstd_hbm, out_hbm, acc, zbuf, vals, didx, sem):
    c = lax.axis_index("c")
    s = lax.axis_index("s")
    zero16 = jnp.zeros((16,), jnp.float32)
    e0 = jnp.where(lax.iota(jnp.int32, 16) == 0, 1.0, 0.0).astype(jnp.float32)

    @pl.loop(0, 640)
    def _(j):
        zbuf[j, :] = zero16

    @pl.loop(0, 128)
    def _(j):
        vals[j, :] = e0

    w = c * 16 + s
    pltpu.sync_copy(dstd_hbm.at[pl.ds(w * 40, 40)], didx)
    pltpu.sync_copy(zbuf, acc.at[pl.ds(s * 640, 640)])
    plsc.subcore_barrier()

    @pl.loop(0, 40)
    def _(i):
        pltpu.async_copy(vals, acc.at[didx.at[i]], sem, add=True)

    @pl.loop(0, 40)
    def _(i):
        pltpu.make_async_copy(vals, acc.at[didx.at[i]], sem).wait()

    plsc.subcore_barrier()
    pltpu.sync_copy(acc.at[pl.ds(s * 640, 640)],
                    out_hbm.at[pl.ds(c * NP + s * 640, 640)])


@functools.partial(
    pl.kernel,
    out_type=jax.ShapeDtypeStruct((2 * NP, H), jnp.float32),
    mesh=_mesh,
    scratch_types=[
        pltpu.VMEM_SHARED((NP, H), jnp.float32),
        pltpu.VMEM((NCH, 128), jnp.int32),
        pltpu.VMEM((NB, 128), jnp.int32),
        pltpu.VMEM((NB, 128, H), jnp.float32),
        pltpu.SemaphoreType.DMA((NB,)),
        pltpu.SemaphoreType.DMA((NB,)),
    ],
)
def _sc_prop(g_hbm, src2_hbm, dstp_hbm, out_hbm, acc, sidx, didx, rows, sem,
             sem2):
    c = lax.axis_index("c")
    s = lax.axis_index("s")
    pltpu.sync_copy(g_hbm.at[pl.ds(c * NP + s * 640, 640)],
                    acc.at[pl.ds(s * 640, 640)])
    w = c * 16 + s
    pltpu.sync_copy(src2_hbm.at[pl.ds(w * NCH, NCH)], sidx)
    plsc.subcore_barrier()

    def _fetch(g, b):
        pltpu.async_copy(g_hbm.at[sidx.at[g]], rows.at[b], sem.at[b])
        pltpu.async_copy(dstp_hbm.at[pl.ds((s * NCH + g) * 128, 128)],
                         didx.at[b], sem2.at[b])

    for b in range(NB):
        _fetch(b, b)

    @pl.loop(0, NCH // NB)
    def _(t):
        for b in range(NB):
            g = t * NB + b
            pltpu.make_async_copy(g_hbm.at[sidx.at[g]], rows.at[b],
                                  sem.at[b]).wait()
            pltpu.make_async_copy(dstp_hbm.at[pl.ds((s * NCH + g) * 128, 128)],
                                  didx.at[b], sem2.at[b]).wait()
            pltpu.sync_copy(rows.at[b], acc.at[didx.at[b]], add=True)
            nxt = g + NB

            @pl.when(nxt < NCH)
            def _():
                _fetch(nxt, b)

    plsc.subcore_barrier()
    pltpu.sync_copy(acc.at[pl.ds(s * 640, 640)],
                    out_hbm.at[pl.ds(c * NP + s * 640, 640)])


def _dis_of(d_blk):
    deg = d_blk[0, :, 0:1] + d_blk[1, :, 0:1] + 1.0
    return lax.rsqrt(deg)


def _mm_first_body(x_ref, w_ref, d_ref, o_ref):
    dis = _dis_of(d_ref[...])
    t = jnp.dot(x_ref[...], w_ref[...], preferred_element_type=jnp.float32)
    g = t * dis
    o_ref[0] = g[:, :H]
    o_ref[1] = g[:, H:]


def _mm_mid_body(p_ref, w_ref, d_ref, o_ref):
    pb = p_ref[...]
    dis = _dis_of(d_ref[...])
    h = jnp.concatenate([pb[0], pb[1]], axis=1)
    h = jnp.maximum(h * dis, 0.0)
    t = jnp.dot(h, w_ref[...], preferred_element_type=jnp.float32)
    g = t * dis
    o_ref[0] = g[:, :H]
    o_ref[1] = g[:, H:]


def _mm_first(x_pad, W, degp):
    return pl.pallas_call(
        _mm_first_body,
        grid=(NP // R,),
        in_specs=[
            pl.BlockSpec((R, D), lambda i: (i, 0)),
            pl.BlockSpec((D, D), lambda i: (0, 0)),
            pl.BlockSpec((2, R, 16), lambda i: (0, i, 0)),
        ],
        out_specs=pl.BlockSpec((2, R, H), lambda i: (0, i, 0)),
        out_shape=jax.ShapeDtypeStruct((2, NP, H), jnp.float32),
    )(x_pad, W, degp)


def _mm_mid(P, W, degp):
    return pl.pallas_call(
        _mm_mid_body,
        grid=(NP // R,),
        in_specs=[
            pl.BlockSpec((2, R, H), lambda i: (0, i, 0)),
            pl.BlockSpec((D, D), lambda i: (0, 0)),
            pl.BlockSpec((2, R, 16), lambda i: (0, i, 0)),
        ],
        out_specs=pl.BlockSpec((2, R, H), lambda i: (0, i, 0)),
        out_shape=jax.ShapeDtypeStruct((2, NP, H), jnp.float32),
    )(P, W, degp)


RP = 2000


def _pool_body(p_ref, d_ref, b_ref, hn_ref, hg_ref, sums, counts):
    i = pl.program_id(0)
    pb = p_ref[...]
    dis = _dis_of(d_ref[...])
    h = jnp.concatenate([pb[0], pb[1]], axis=1)
    h = jnp.maximum(h * dis, 0.0)
    hn_ref[...] = h
    bb = b_ref[...]
    gid = lax.broadcasted_iota(jnp.int32, (RP, NG), 1)
    mask = (bb == gid).astype(jnp.float32)
    dn = (((0,), (0,)), ((), ()))
    s_blk = lax.dot_general(mask, h, dn, preferred_element_type=jnp.float32)
    ones = jnp.ones((RP, H), jnp.float32)
    c_blk = lax.dot_general(mask, ones, dn, preferred_element_type=jnp.float32)

    @pl.when(i == 0)
    def _():
        sums[...] = s_blk
        counts[...] = c_blk

    @pl.when(i > 0)
    def _():
        sums[...] += s_blk
        counts[...] += c_blk

    @pl.when(i == NK // RP - 1)
    def _():
        cnt = jnp.maximum(counts[...][:, 0:1], 1.0)
        hg_ref[...] = sums[...] / cnt


def _pool(P, degp, batch_col):
    return pl.pallas_call(
        _pool_body,
        grid=(NK // RP,),
        in_specs=[
            pl.BlockSpec((2, RP, H), lambda i: (0, i, 0)),
            pl.BlockSpec((2, RP, 16), lambda i: (0, i, 0)),
            pl.BlockSpec((RP, 1), lambda i: (i, 0)),
        ],
        out_specs=[
            pl.BlockSpec((RP, D), lambda i: (i, 0)),
            pl.BlockSpec((NG, D), lambda i: (0, 0)),
        ],
        out_shape=[
            jax.ShapeDtypeStruct((NK, D), jnp.float32),
            jax.ShapeDtypeStruct((NG, D), jnp.float32),
        ],
        scratch_shapes=[
            pltpu.VMEM((NG, D), jnp.float32),
            pltpu.VMEM((NG, H), jnp.float32),
        ],
    )(P, degp, batch_col)


def kernel(x, edge_index, edge_weight, batch, W0, W1, W2):
    src = edge_index[0]
    dst = edge_index[1]

    e = jnp.arange(E, EP, dtype=jnp.int32)
    src_p = jnp.concatenate([src, e % NP])
    src2 = jnp.concatenate([src_p, src_p + NP])
    src2 = src2.reshape(2 * 16 * NCH, 128)
    dstd = jnp.concatenate([dst, NK + e % (NP - NK)])

    x_pad = jnp.pad(x, ((0, NP - NK), (0, 0)))
    batch_col = batch.reshape(NK, 1)

    degp = _sc_deg(dstd.reshape(32 * 40, 128)).reshape(2, NP, 16)
    g = _mm_first(x_pad, W0, degp)
    P = _sc_prop(g.reshape(2 * NP, H), src2, dstd).reshape(2, NP, H)
    g = _mm_mid(P, W1, degp)
    P = _sc_prop(g.reshape(2 * NP, H), src2, dstd).reshape(2, NP, H)
    g = _mm_mid(P, W2, degp)
    P = _sc_prop(g.reshape(2 * NP, H), src2, dstd).reshape(2, NP, H)
    h_node, h_graph = _pool(P, degp, batch_col)
    return h_node, h_graph
